# baseline (speedup 1.0000x reference)
import jax
import jax.numpy as jnp
from jax.experimental import pallas as pl
from jax.experimental.pallas import tpu as pltpu

_N_SUBS = 64
_D_MID = 64
_D_OUT = 64
_ST_W = 256


def _pass1_body(x_ref, sub_ref, st_ref):
    x = x_ref[...]
    b, d_in = x.shape
    xh = x.astype(jnp.bfloat16)
    sub = sub_ref[0, 0, :]
    segT = jax.lax.broadcasted_iota(jnp.int32, (_N_SUBS, b), 0)
    maskT = sub[None, :] == segT
    mseg = jax.lax.dot_general(
        maskT.astype(jnp.bfloat16), xh, (((1,), (0,)), ((), ())),
        preferred_element_type=jnp.float32)
    cnt = jnp.sum(maskT.astype(jnp.float32), axis=1, keepdims=True)
    r0 = jnp.sum(x[0:1, :], axis=1, keepdims=True)

    st_ref[0, :, :d_in] = mseg
    st_ref[0, :, d_in:d_in + 1] = cnt
    st_ref[0, :, d_in + 2:] = jnp.zeros((_N_SUBS, _ST_W - d_in - 2), jnp.float32)
    st_ref[0, 0:1, d_in + 1:d_in + 2] = r0
    st_ref[0, 1:, d_in + 1:d_in + 2] = jnp.zeros((_N_SUBS - 1, 1), jnp.float32)


def _pass2_body(g_ref, l_ref, x_ref, sub_ref, st_ref, out_ref):
    x = x_ref[...]
    b, d_in = x.shape
    st = jnp.sum(st_ref[...], axis=0)
    S = jnp.sum(st[:, :d_in], axis=1)
    C = st[:, d_in]

    xh = x.astype(jnp.bfloat16)
    xl = (x - xh.astype(jnp.float32)).astype(jnp.bfloat16)
    ones = jnp.ones((d_in, _D_OUT), jnp.bfloat16)
    dot = lambda a, c: jax.lax.dot_general(
        a, c, (((1,), (0,)), ((), ())), preferred_element_type=jnp.float32)
    rB = dot(xh, ones) + dot(xl, ones)

    r0 = st_ref[0, 0, d_in + 1]
    m = jnp.where(C > 0, S / jnp.maximum(C, 1.0), r0)
    gamma = g_ref[0, 0]
    t = jnp.maximum(gamma * m, 0.0) * _D_MID
    t2 = jnp.broadcast_to(t[:, None], (_N_SUBS, _N_SUBS))
    t2 = t2.astype(jnp.bfloat16)

    sub = sub_ref[0, 0, :]
    seg = jax.lax.broadcasted_iota(jnp.int32, (b, _N_SUBS), 1)
    mask = (sub[:, None] == seg).astype(jnp.bfloat16)
    gB = dot(mask, t2)
    lam = l_ref[0, 0]
    out_ref[...] = jnp.maximum(lam * (rB + gB), 0.0)


def kernel(x, sub, Gamma, Lambda):
    n, d_in = x.shape
    B = 16000
    nb = n // B
    sub3 = sub.reshape(nb, 1, B)
    gv = jnp.broadcast_to(Gamma.reshape(1, 1), (8, 128))
    lv = jnp.broadcast_to(Lambda.reshape(1, 1), (8, 128))
    par = pltpu.CompilerParams(dimension_semantics=("parallel",))

    st = pl.pallas_call(
        _pass1_body,
        grid=(nb,),
        in_specs=[
            pl.BlockSpec((B, d_in), lambda i: (i, 0)),
            pl.BlockSpec((1, 1, B), lambda i: (i, 0, 0)),
        ],
        out_specs=pl.BlockSpec((1, _N_SUBS, _ST_W), lambda i: (i, 0, 0)),
        out_shape=jax.ShapeDtypeStruct((nb, _N_SUBS, _ST_W), jnp.float32),
        compiler_params=par,
    )(x, sub3)

    out = pl.pallas_call(
        _pass2_body,
        grid=(nb,),
        in_specs=[
            pl.BlockSpec((8, 128), lambda i: (0, 0)),
            pl.BlockSpec((8, 128), lambda i: (0, 0)),
            pl.BlockSpec((B, d_in), lambda i: (i, 0)),
            pl.BlockSpec((1, 1, B), lambda i: (i, 0, 0)),
            pl.BlockSpec((nb, _N_SUBS, _ST_W), lambda i: (0, 0, 0)),
        ],
        out_specs=pl.BlockSpec((B, _D_OUT), lambda i: (i, 0)),
        out_shape=jax.ShapeDtypeStruct((n, _D_OUT), jnp.float32),
        compiler_params=par,
    )(gv, lv, x, sub3, st)
    return out
